# baseline (speedup 1.0000x reference)
import jax
import jax.numpy as jnp
from jax.experimental import pallas as pl
from jax.experimental.pallas import tpu as pltpu

B = 1024
IN_DIM = 512
D_MODEL = 1024
D_FF = 4096
N_LAYERS = 3
OUT_DIM = 128

BM = 512
FF_BLK = 2048
NB = B // BM
NFF = D_FF // FF_BLK

BMH = 512
FFH = 2048
NBH = B // BMH
NFH = D_FF // FFH


def _dot(a, b, dims):
    return jax.lax.dot_general(a, b, (dims, ((), ())),
                               preferred_element_type=jnp.float32)


def _dotb(a, b, dims):
    return jax.lax.dot_general(a.astype(jnp.bfloat16), b.astype(jnp.bfloat16),
                               (dims, ((), ())),
                               preferred_element_type=jnp.float32)


def _gelu_exact(x):
    return 0.5 * x * (1.0 + jax.lax.erf(x * 0.7071067811865476))


def _ln(a, g, b, eps=1e-5):
    m = jnp.mean(a, axis=-1, keepdims=True)
    d = a - m
    v = jnp.mean(d * d, axis=-1, keepdims=True)
    return d * jax.lax.rsqrt(v + eps) * g + b


def _fold_body(Wv_ref, Wo_ref, bv_ref, bo_ref, WvoT_ref, bvo_ref):
    WvoT_ref[...] = _dot(Wv_ref[...], Wo_ref[...], ((0,), (1,)))
    bvo_ref[...] = _dot(bv_ref[...], Wo_ref[...], ((1,), (1,))) + bo_ref[...]


def _main_body(x_ref, Win_ref, bin_ref, WvoT_ref, bvo_ref,
               W1_ref, b1_ref, W2_ref, b2_ref,
               g1_ref, be1_ref, g2_ref, be2_ref, gf_ref, bf_ref,
               out_ref, h_s, h1_s, acc_s):
    l = pl.program_id(1)
    c = pl.program_id(2)

    @pl.when((l == 0) & (c == 0))
    def _():
        h_s[...] = _dotb(x_ref[...], Win_ref[...], ((1,), (1,))) + bin_ref[...]

    @pl.when(c == 0)
    def _():
        h = h_s[...]
        a = h + _dotb(h, WvoT_ref[...], ((1,), (0,))) + bvo_ref[...]
        h1_s[...] = _ln(a, g1_ref[0], be1_ref[0])

    h1 = h1_s[...]
    gc = _gelu_exact(_dotb(h1, W1_ref[0], ((1,), (1,))) + b1_ref[0])
    part = _dotb(gc, W2_ref[0], ((1,), (1,)))

    @pl.when(c == 0)
    def _():
        acc_s[...] = part

    @pl.when(c > 0)
    def _():
        acc_s[...] += part

    @pl.when(c == NFF - 1)
    def _():
        a2 = h1_s[...] + acc_s[...] + b2_ref[0]
        h_s[...] = _ln(a2, g2_ref[0], be2_ref[0])

    @pl.when((c == NFF - 1) & (l == N_LAYERS - 1))
    def _():
        out_ref[...] = _ln(h_s[...], gf_ref[...], bf_ref[...])


def _head_body(hf_ref, W1_ref, b1_ref, W2_ref, b2_ref, out_ref):
    c = pl.program_id(1)
    g = _gelu_exact(_dotb(hf_ref[...], W1_ref[...], ((1,), (1,))) + b1_ref[...])
    part = _dotb(g, W2_ref[...], ((1,), (1,)))

    @pl.when(c == 0)
    def _():
        out_ref[...] = part + b2_ref[...]

    @pl.when(c > 0)
    def _():
        out_ref[...] += part


def kernel(x, W_in, b_in, Wq, bq, Wk, bk, Wv, bv, Wo, bo,
           W1, b1, W2, b2, g1, be1, g2, be2, gf, bf,
           Wout1, bout1, Wout2, bout2):
    f32 = jnp.float32
    bv2 = bv.reshape(1, D_MODEL)
    bo2 = bo.reshape(1, D_MODEL)
    bin2 = b_in.reshape(1, D_MODEL)
    b1r = b1.reshape(N_LAYERS, 1, D_FF)
    b2r = b2.reshape(N_LAYERS, 1, D_MODEL)
    g1r = g1.reshape(N_LAYERS, 1, D_MODEL)
    be1r = be1.reshape(N_LAYERS, 1, D_MODEL)
    g2r = g2.reshape(N_LAYERS, 1, D_MODEL)
    be2r = be2.reshape(N_LAYERS, 1, D_MODEL)
    gf2 = gf.reshape(1, D_MODEL)
    bf2 = bf.reshape(1, D_MODEL)
    bout1r = bout1.reshape(1, D_FF)
    bout2r = bout2.reshape(1, OUT_DIM)

    WvoT, bvo = pl.pallas_call(
        _fold_body,
        out_shape=[jax.ShapeDtypeStruct((D_MODEL, D_MODEL), f32),
                   jax.ShapeDtypeStruct((1, D_MODEL), f32)],
    )(Wv, Wo, bv2, bo2)

    hf = pl.pallas_call(
        _main_body,
        grid=(NB, N_LAYERS, NFF),
        in_specs=[
            pl.BlockSpec((BM, IN_DIM), lambda b, l, c: (b, 0)),
            pl.BlockSpec((D_MODEL, IN_DIM), lambda b, l, c: (0, 0)),
            pl.BlockSpec((1, D_MODEL), lambda b, l, c: (0, 0)),
            pl.BlockSpec((D_MODEL, D_MODEL), lambda b, l, c: (0, 0)),
            pl.BlockSpec((1, D_MODEL), lambda b, l, c: (0, 0)),
            pl.BlockSpec((1, FF_BLK, D_MODEL), lambda b, l, c: (l, c, 0)),
            pl.BlockSpec((1, 1, FF_BLK), lambda b, l, c: (l, 0, c)),
            pl.BlockSpec((1, D_MODEL, FF_BLK), lambda b, l, c: (l, 0, c)),
            pl.BlockSpec((1, 1, D_MODEL), lambda b, l, c: (l, 0, 0)),
            pl.BlockSpec((1, 1, D_MODEL), lambda b, l, c: (l, 0, 0)),
            pl.BlockSpec((1, 1, D_MODEL), lambda b, l, c: (l, 0, 0)),
            pl.BlockSpec((1, 1, D_MODEL), lambda b, l, c: (l, 0, 0)),
            pl.BlockSpec((1, 1, D_MODEL), lambda b, l, c: (l, 0, 0)),
            pl.BlockSpec((1, D_MODEL), lambda b, l, c: (0, 0)),
            pl.BlockSpec((1, D_MODEL), lambda b, l, c: (0, 0)),
        ],
        out_specs=pl.BlockSpec((BM, D_MODEL), lambda b, l, c: (b, 0)),
        out_shape=jax.ShapeDtypeStruct((B, D_MODEL), f32),
        scratch_shapes=[pltpu.VMEM((BM, D_MODEL), f32)] * 3,
        compiler_params=pltpu.CompilerParams(
            dimension_semantics=("parallel", "arbitrary", "arbitrary")),
    )(x, W_in, bin2, WvoT, bvo, W1, b1r, W2, b2r,
      g1r, be1r, g2r, be2r, gf2, bf2)

    out = pl.pallas_call(
        _head_body,
        grid=(NBH, NFH),
        in_specs=[
            pl.BlockSpec((BMH, D_MODEL), lambda b, c: (b, 0)),
            pl.BlockSpec((FFH, D_MODEL), lambda b, c: (c, 0)),
            pl.BlockSpec((1, FFH), lambda b, c: (0, c)),
            pl.BlockSpec((OUT_DIM, FFH), lambda b, c: (0, c)),
            pl.BlockSpec((1, OUT_DIM), lambda b, c: (0, 0)),
        ],
        out_specs=pl.BlockSpec((BMH, OUT_DIM), lambda b, c: (b, 0)),
        out_shape=jax.ShapeDtypeStruct((B, OUT_DIM), f32),
        compiler_params=pltpu.CompilerParams(
            dimension_semantics=("parallel", "arbitrary")),
    )(hf, Wout1, bout1r, Wout2, bout2r)

    return out


# speedup vs baseline: 1.0403x; 1.0403x over previous
import jax
import jax.numpy as jnp
from jax.experimental import pallas as pl
from jax.experimental.pallas import tpu as pltpu

B = 1024
IN_DIM = 512
D_MODEL = 1024
D_FF = 4096
N_LAYERS = 3
OUT_DIM = 128

BM = 1024
FF_BLK = 1024
NB = B // BM
NFF = D_FF // FF_BLK

BMH = 512
FFH = 2048
NBH = B // BMH
NFH = D_FF // FFH


def _dot(a, b, dims):
    return jax.lax.dot_general(a, b, (dims, ((), ())),
                               preferred_element_type=jnp.float32)


def _dotb(a, b, dims):
    return jax.lax.dot_general(a.astype(jnp.bfloat16), b.astype(jnp.bfloat16),
                               (dims, ((), ())),
                               preferred_element_type=jnp.float32)


def _gelu_exact(x):
    return 0.5 * x * (1.0 + jax.lax.erf(x * 0.7071067811865476))


def _ln(a, g, b, eps=1e-5):
    m = jnp.mean(a, axis=-1, keepdims=True)
    d = a - m
    v = jnp.mean(d * d, axis=-1, keepdims=True)
    return d * jax.lax.rsqrt(v + eps) * g + b


def _fold_body(Wv_ref, Wo_ref, bv_ref, bo_ref, WvoT_ref, bvo_ref):
    WvoT_ref[...] = _dot(Wv_ref[...], Wo_ref[...], ((0,), (1,)))
    bvo_ref[...] = _dot(bv_ref[...], Wo_ref[...], ((1,), (1,))) + bo_ref[...]


def _main_body(x_ref, Win_ref, bin_ref, WvoT_ref, bvo_ref,
               W1_ref, b1_ref, W2_ref, b2_ref,
               g1_ref, be1_ref, g2_ref, be2_ref, gf_ref, bf_ref,
               out_ref, h_s, h1_s, acc_s):
    l = pl.program_id(1)
    c = pl.program_id(2)

    @pl.when((l == 0) & (c == 0))
    def _():
        h_s[...] = _dotb(x_ref[...], Win_ref[...], ((1,), (1,))) + bin_ref[...]

    @pl.when(c == 0)
    def _():
        h = h_s[...]
        a = h + _dotb(h, WvoT_ref[...], ((1,), (0,))) + bvo_ref[...]
        h1_s[...] = _ln(a, g1_ref[0], be1_ref[0])

    h1 = h1_s[...]
    gc = _gelu_exact(_dotb(h1, W1_ref[0], ((1,), (1,))) + b1_ref[0])
    part = _dotb(gc, W2_ref[0], ((1,), (1,)))

    @pl.when(c == 0)
    def _():
        acc_s[...] = part

    @pl.when(c > 0)
    def _():
        acc_s[...] += part

    @pl.when(c == NFF - 1)
    def _():
        a2 = h1_s[...] + acc_s[...] + b2_ref[0]
        h_s[...] = _ln(a2, g2_ref[0], be2_ref[0])

    @pl.when((c == NFF - 1) & (l == N_LAYERS - 1))
    def _():
        out_ref[...] = _ln(h_s[...], gf_ref[...], bf_ref[...])


def _head_body(hf_ref, W1_ref, b1_ref, W2_ref, b2_ref, out_ref):
    c = pl.program_id(1)
    g = _gelu_exact(_dotb(hf_ref[...], W1_ref[...], ((1,), (1,))) + b1_ref[...])
    part = _dotb(g, W2_ref[...], ((1,), (1,)))

    @pl.when(c == 0)
    def _():
        out_ref[...] = part + b2_ref[...]

    @pl.when(c > 0)
    def _():
        out_ref[...] += part


def kernel(x, W_in, b_in, Wq, bq, Wk, bk, Wv, bv, Wo, bo,
           W1, b1, W2, b2, g1, be1, g2, be2, gf, bf,
           Wout1, bout1, Wout2, bout2):
    f32 = jnp.float32
    bv2 = bv.reshape(1, D_MODEL)
    bo2 = bo.reshape(1, D_MODEL)
    bin2 = b_in.reshape(1, D_MODEL)
    b1r = b1.reshape(N_LAYERS, 1, D_FF)
    b2r = b2.reshape(N_LAYERS, 1, D_MODEL)
    g1r = g1.reshape(N_LAYERS, 1, D_MODEL)
    be1r = be1.reshape(N_LAYERS, 1, D_MODEL)
    g2r = g2.reshape(N_LAYERS, 1, D_MODEL)
    be2r = be2.reshape(N_LAYERS, 1, D_MODEL)
    gf2 = gf.reshape(1, D_MODEL)
    bf2 = bf.reshape(1, D_MODEL)
    bout1r = bout1.reshape(1, D_FF)
    bout2r = bout2.reshape(1, OUT_DIM)

    WvoT, bvo = pl.pallas_call(
        _fold_body,
        out_shape=[jax.ShapeDtypeStruct((D_MODEL, D_MODEL), f32),
                   jax.ShapeDtypeStruct((1, D_MODEL), f32)],
    )(Wv, Wo, bv2, bo2)

    hf = pl.pallas_call(
        _main_body,
        grid=(NB, N_LAYERS, NFF),
        in_specs=[
            pl.BlockSpec((BM, IN_DIM), lambda b, l, c: (b, 0)),
            pl.BlockSpec((D_MODEL, IN_DIM), lambda b, l, c: (0, 0)),
            pl.BlockSpec((1, D_MODEL), lambda b, l, c: (0, 0)),
            pl.BlockSpec((D_MODEL, D_MODEL), lambda b, l, c: (0, 0)),
            pl.BlockSpec((1, D_MODEL), lambda b, l, c: (0, 0)),
            pl.BlockSpec((1, FF_BLK, D_MODEL), lambda b, l, c: (l, c, 0)),
            pl.BlockSpec((1, 1, FF_BLK), lambda b, l, c: (l, 0, c)),
            pl.BlockSpec((1, D_MODEL, FF_BLK), lambda b, l, c: (l, 0, c)),
            pl.BlockSpec((1, 1, D_MODEL), lambda b, l, c: (l, 0, 0)),
            pl.BlockSpec((1, 1, D_MODEL), lambda b, l, c: (l, 0, 0)),
            pl.BlockSpec((1, 1, D_MODEL), lambda b, l, c: (l, 0, 0)),
            pl.BlockSpec((1, 1, D_MODEL), lambda b, l, c: (l, 0, 0)),
            pl.BlockSpec((1, 1, D_MODEL), lambda b, l, c: (l, 0, 0)),
            pl.BlockSpec((1, D_MODEL), lambda b, l, c: (0, 0)),
            pl.BlockSpec((1, D_MODEL), lambda b, l, c: (0, 0)),
        ],
        out_specs=pl.BlockSpec((BM, D_MODEL), lambda b, l, c: (b, 0)),
        out_shape=jax.ShapeDtypeStruct((B, D_MODEL), f32),
        scratch_shapes=[pltpu.VMEM((BM, D_MODEL), f32)] * 3,
        compiler_params=pltpu.CompilerParams(
            dimension_semantics=("parallel", "arbitrary", "arbitrary")),
    )(x, W_in, bin2, WvoT, bvo, W1, b1r, W2, b2r,
      g1r, be1r, g2r, be2r, gf2, bf2)

    out = pl.pallas_call(
        _head_body,
        grid=(NBH, NFH),
        in_specs=[
            pl.BlockSpec((BMH, D_MODEL), lambda b, c: (b, 0)),
            pl.BlockSpec((FFH, D_MODEL), lambda b, c: (c, 0)),
            pl.BlockSpec((1, FFH), lambda b, c: (0, c)),
            pl.BlockSpec((OUT_DIM, FFH), lambda b, c: (0, c)),
            pl.BlockSpec((1, OUT_DIM), lambda b, c: (0, 0)),
        ],
        out_specs=pl.BlockSpec((BMH, OUT_DIM), lambda b, c: (b, 0)),
        out_shape=jax.ShapeDtypeStruct((B, OUT_DIM), f32),
        compiler_params=pltpu.CompilerParams(
            dimension_semantics=("parallel", "arbitrary")),
    )(hf, Wout1, bout1r, Wout2, bout2r)

    return out


# speedup vs baseline: 1.0443x; 1.0039x over previous
import jax
import jax.numpy as jnp
from jax.experimental import pallas as pl
from jax.experimental.pallas import tpu as pltpu

B = 1024
IN_DIM = 512
D_MODEL = 1024
D_FF = 4096
N_LAYERS = 3
OUT_DIM = 128

BM = 1024
FF_BLK = 1024
NB = B // BM
NFF = D_FF // FF_BLK

BMH = 512
FFH = 2048
NBH = B // BMH
NFH = D_FF // FFH


def _dot(a, b, dims):
    return jax.lax.dot_general(a, b, (dims, ((), ())),
                               preferred_element_type=jnp.float32)


def _dotb(a, b, dims):
    return jax.lax.dot_general(a.astype(jnp.bfloat16), b.astype(jnp.bfloat16),
                               (dims, ((), ())),
                               preferred_element_type=jnp.float32)


def _gelu_exact(x):
    return 0.5 * x * (1.0 + jax.lax.erf(x * 0.7071067811865476))


def _ln(a, g, b, eps=1e-5):
    m = jnp.mean(a, axis=-1, keepdims=True)
    d = a - m
    v = jnp.mean(d * d, axis=-1, keepdims=True)
    return d * jax.lax.rsqrt(v + eps) * g + b


def _fold_body(Wv_ref, Wo_ref, bv_ref, bo_ref, WvoT_ref, bvo_ref):
    WvoT_ref[...] = _dot(Wv_ref[...], Wo_ref[...], ((0,), (1,)))
    bvo_ref[...] = _dot(bv_ref[...], Wo_ref[...], ((1,), (1,))) + bo_ref[...]


def _main_body(x_ref, Win_ref, bin_ref, WvoT_ref, bvo_ref,
               W1_ref, b1_ref, W2_ref, b2_ref,
               g1_ref, be1_ref, g2_ref, be2_ref, gf_ref, bf_ref,
               out_ref, h_s, h1_s, acc_s):
    l = pl.program_id(1)
    c = pl.program_id(2)

    @pl.when((l == 0) & (c == 0))
    def _():
        h_s[...] = _dotb(x_ref[...], Win_ref[...], ((1,), (1,))) + bin_ref[...]

    @pl.when(c == 0)
    def _():
        h = h_s[...]
        a = h + _dotb(h, WvoT_ref[...], ((1,), (0,))) + bvo_ref[...]
        h1_s[...] = _ln(a, g1_ref[0], be1_ref[0])

    h1 = h1_s[...]
    gc = _gelu_exact(_dot(h1, W1_ref[0], ((1,), (1,))) + b1_ref[0])
    part = _dot(gc, W2_ref[0], ((1,), (1,)))

    @pl.when(c == 0)
    def _():
        acc_s[...] = part

    @pl.when(c > 0)
    def _():
        acc_s[...] += part

    @pl.when(c == NFF - 1)
    def _():
        a2 = h1_s[...] + acc_s[...] + b2_ref[0]
        h_s[...] = _ln(a2, g2_ref[0], be2_ref[0])

    @pl.when((c == NFF - 1) & (l == N_LAYERS - 1))
    def _():
        out_ref[...] = _ln(h_s[...], gf_ref[...], bf_ref[...])


def _head_body(hf_ref, W1_ref, b1_ref, W2_ref, b2_ref, out_ref):
    c = pl.program_id(1)
    g = _gelu_exact(_dot(hf_ref[...], W1_ref[...], ((1,), (1,))) + b1_ref[...])
    part = _dot(g, W2_ref[...], ((1,), (1,)))

    @pl.when(c == 0)
    def _():
        out_ref[...] = part + b2_ref[...]

    @pl.when(c > 0)
    def _():
        out_ref[...] += part


def kernel(x, W_in, b_in, Wq, bq, Wk, bk, Wv, bv, Wo, bo,
           W1, b1, W2, b2, g1, be1, g2, be2, gf, bf,
           Wout1, bout1, Wout2, bout2):
    f32 = jnp.float32
    bv2 = bv.reshape(1, D_MODEL)
    bo2 = bo.reshape(1, D_MODEL)
    bin2 = b_in.reshape(1, D_MODEL)
    b1r = b1.reshape(N_LAYERS, 1, D_FF)
    b2r = b2.reshape(N_LAYERS, 1, D_MODEL)
    g1r = g1.reshape(N_LAYERS, 1, D_MODEL)
    be1r = be1.reshape(N_LAYERS, 1, D_MODEL)
    g2r = g2.reshape(N_LAYERS, 1, D_MODEL)
    be2r = be2.reshape(N_LAYERS, 1, D_MODEL)
    gf2 = gf.reshape(1, D_MODEL)
    bf2 = bf.reshape(1, D_MODEL)
    bout1r = bout1.reshape(1, D_FF)
    bout2r = bout2.reshape(1, OUT_DIM)

    WvoT, bvo = pl.pallas_call(
        _fold_body,
        out_shape=[jax.ShapeDtypeStruct((D_MODEL, D_MODEL), f32),
                   jax.ShapeDtypeStruct((1, D_MODEL), f32)],
    )(Wv, Wo, bv2, bo2)

    hf = pl.pallas_call(
        _main_body,
        grid=(NB, N_LAYERS, NFF),
        in_specs=[
            pl.BlockSpec((BM, IN_DIM), lambda b, l, c: (b, 0)),
            pl.BlockSpec((D_MODEL, IN_DIM), lambda b, l, c: (0, 0)),
            pl.BlockSpec((1, D_MODEL), lambda b, l, c: (0, 0)),
            pl.BlockSpec((D_MODEL, D_MODEL), lambda b, l, c: (0, 0)),
            pl.BlockSpec((1, D_MODEL), lambda b, l, c: (0, 0)),
            pl.BlockSpec((1, FF_BLK, D_MODEL), lambda b, l, c: (l, c, 0)),
            pl.BlockSpec((1, 1, FF_BLK), lambda b, l, c: (l, 0, c)),
            pl.BlockSpec((1, D_MODEL, FF_BLK), lambda b, l, c: (l, 0, c)),
            pl.BlockSpec((1, 1, D_MODEL), lambda b, l, c: (l, 0, 0)),
            pl.BlockSpec((1, 1, D_MODEL), lambda b, l, c: (l, 0, 0)),
            pl.BlockSpec((1, 1, D_MODEL), lambda b, l, c: (l, 0, 0)),
            pl.BlockSpec((1, 1, D_MODEL), lambda b, l, c: (l, 0, 0)),
            pl.BlockSpec((1, 1, D_MODEL), lambda b, l, c: (l, 0, 0)),
            pl.BlockSpec((1, D_MODEL), lambda b, l, c: (0, 0)),
            pl.BlockSpec((1, D_MODEL), lambda b, l, c: (0, 0)),
        ],
        out_specs=pl.BlockSpec((BM, D_MODEL), lambda b, l, c: (b, 0)),
        out_shape=jax.ShapeDtypeStruct((B, D_MODEL), f32),
        scratch_shapes=[pltpu.VMEM((BM, D_MODEL), f32)] * 3,
        compiler_params=pltpu.CompilerParams(
            dimension_semantics=("parallel", "arbitrary", "arbitrary")),
    )(x, W_in, bin2, WvoT, bvo, W1, b1r, W2, b2r,
      g1r, be1r, g2r, be2r, gf2, bf2)

    out = pl.pallas_call(
        _head_body,
        grid=(NBH, NFH),
        in_specs=[
            pl.BlockSpec((BMH, D_MODEL), lambda b, c: (b, 0)),
            pl.BlockSpec((FFH, D_MODEL), lambda b, c: (c, 0)),
            pl.BlockSpec((1, FFH), lambda b, c: (0, c)),
            pl.BlockSpec((OUT_DIM, FFH), lambda b, c: (0, c)),
            pl.BlockSpec((1, OUT_DIM), lambda b, c: (0, 0)),
        ],
        out_specs=pl.BlockSpec((BMH, OUT_DIM), lambda b, c: (b, 0)),
        out_shape=jax.ShapeDtypeStruct((B, OUT_DIM), f32),
        compiler_params=pltpu.CompilerParams(
            dimension_semantics=("parallel", "arbitrary")),
    )(hf, Wout1, bout1r, Wout2, bout2r)

    return out
